# baseline (speedup 1.0000x reference)
import functools

import numpy as np
import jax
import jax.numpy as jnp
from jax.experimental import pallas as pl

_B = 256
_N = 1024
_NUM_MASK = 768
_TAIL = _N - _NUM_MASK
_R = 64


def _threefry_bits_u32(n):
    k0 = jnp.uint32(0)
    k1 = jnp.uint32(1)
    k2 = jnp.uint32(0x1BD11BDA) ^ k0 ^ k1
    ks = (k0, k1, k2)
    rot = ((13, 15, 26, 6), (17, 29, 16, 24))
    x0 = jnp.zeros_like(n) + k0
    x1 = n + k1
    for i in range(5):
        for r in rot[i % 2]:
            x0 = x0 + x1
            x1 = (x1 << jnp.uint32(r)) | (x1 >> jnp.uint32(32 - r))
            x1 = x0 ^ x1
        x0 = x0 + ks[(i + 1) % 3]
        x1 = x1 + ks[(i + 2) % 3] + jnp.uint32(i + 1)
    return x0 ^ x1


def _body(ones8_ref, out_ref):
    step = pl.program_id(0)

    j = jax.lax.broadcasted_iota(jnp.uint32, (_R, _N), 1)
    si = jax.lax.broadcasted_iota(jnp.uint32, (_R, _N), 0)
    n = (step * (_R * _N)).astype(jnp.uint32) + si * jnp.uint32(_N) + j
    bits = _threefry_bits_u32(n)

    k = (bits & jnp.uint32(0xFFFFFE00)) | (j >> jnp.uint32(1))
    k = k ^ jnp.uint32(0x80000000)
    kxi = jax.lax.bitcast_convert_type(k, jnp.int32)

    tail = kxi[:, _NUM_MASK:]
    tt_i = jnp.transpose(tail, (1, 0))

    ch_iota = jax.lax.broadcasted_iota(jnp.int32, (_TAIL, 32), 1)

    for rr in range(_R):
        t_col = jax.lax.slice(tt_i, (0, rr), (_TAIL, rr + 1))
        krow = kxi[rr:rr + 1, :]
        cmpf = (krow < t_col).astype(jnp.float32)
        r_w = jax.lax.dot_general(
            cmpf, ones8_ref[...], (((1,), (0,)), ((), ())),
            preferred_element_type=jnp.float32)
        r_i = jax.lax.slice(r_w, (0, 0), (_TAIL, 1)).astype(jnp.int32)
        rhi = r_i >> 5
        rlo = r_i & 31
        u = (rhi == ch_iota).astype(jnp.float32)
        v = (rlo == ch_iota).astype(jnp.float32)
        s = jax.lax.dot_general(
            u, v, (((0,), (0,)), ((), ())),
            preferred_element_type=jnp.float32)
        out_ref[rr] = s


@functools.partial(jax.jit, static_argnames=("interpret",))
def _random_mask(interpret=False):
    return pl.pallas_call(
        _body,
        grid=(_B // _R,),
        in_specs=[pl.BlockSpec((_N, 8), lambda r: (0, 0))],
        out_specs=pl.BlockSpec((_R, 32, 32), lambda r: (r, 0, 0)),
        out_shape=jax.ShapeDtypeStruct((_B, 32, 32), jnp.float32),
        interpret=interpret,
    )(jnp.asarray(np.ones((_N, 8), np.float32))).reshape(_B, _N) == 0.0


def kernel(x):
    assert x.shape[0] == _B
    return _random_mask()


# speedup vs baseline: 1.6798x; 1.6798x over previous
import functools

import jax
import jax.numpy as jnp
from jax.experimental import pallas as pl

_B = 256
_N = 1024
_NUM_MASK = 768
_TAIL = _N - _NUM_MASK
_R = 64


def _threefry_bits_u32(n):
    k0 = jnp.uint32(0)
    k1 = jnp.uint32(1)
    k2 = jnp.uint32(0x1BD11BDA) ^ k0 ^ k1
    ks = (k0, k1, k2)
    rot = ((13, 15, 26, 6), (17, 29, 16, 24))
    x0 = jnp.zeros_like(n) + k0
    x1 = n + k1
    for i in range(5):
        for r in rot[i % 2]:
            x0 = x0 + x1
            x1 = (x1 << jnp.uint32(r)) | (x1 >> jnp.uint32(32 - r))
            x1 = x0 ^ x1
        x0 = x0 + ks[(i + 1) % 3]
        x1 = x1 + ks[(i + 2) % 3] + jnp.uint32(i + 1)
    return x0 ^ x1


def _body(out_ref):
    step = pl.program_id(0)

    j = jax.lax.broadcasted_iota(jnp.uint32, (_R, _N), 1)
    si = jax.lax.broadcasted_iota(jnp.uint32, (_R, _N), 0)
    n = (step * (_R * _N)).astype(jnp.uint32) + si * jnp.uint32(_N) + j
    bits = _threefry_bits_u32(n)

    k = (bits & jnp.uint32(0xFFFFFE00)) | (j >> jnp.uint32(1))
    k = k ^ jnp.uint32(0x80000000)
    kxi = jax.lax.bitcast_convert_type(k, jnp.int32)

    tail = kxi[:, _NUM_MASK:]
    tt_i = jnp.transpose(tail, (1, 0))

    ch_iota = jax.lax.broadcasted_iota(jnp.int32, (_TAIL, 32), 1)

    for rr in range(_R):
        t_col = jax.lax.slice(tt_i, (0, rr), (_TAIL, rr + 1))
        krow = kxi[rr:rr + 1, :]
        cmpf = (krow < t_col).astype(jnp.float32)
        r_f = jnp.sum(cmpf, axis=1, keepdims=True)
        r_i = r_f.astype(jnp.int32)
        rhi = r_i >> 5
        rlo = r_i & 31
        u = (rhi == ch_iota).astype(jnp.float32)
        v = (rlo == ch_iota).astype(jnp.float32)
        s = jax.lax.dot_general(
            u, v, (((0,), (0,)), ((), ())),
            preferred_element_type=jnp.float32)
        out_ref[rr] = s


@functools.partial(jax.jit, static_argnames=("interpret",))
def _random_mask(interpret=False):
    return pl.pallas_call(
        _body,
        grid=(_B // _R,),
        in_specs=[],
        out_specs=pl.BlockSpec((_R, 32, 32), lambda r: (r, 0, 0)),
        out_shape=jax.ShapeDtypeStruct((_B, 32, 32), jnp.float32),
        interpret=interpret,
    )().reshape(_B, _N) == 0.0


def kernel(x):
    assert x.shape[0] == _B
    return _random_mask()
